# Initial kernel scaffold; baseline (speedup 1.0000x reference)
import jax
import jax.numpy as jnp
from jax.experimental import pallas as pl


def kernel(x, W1, b1, W2, b2, W3, b3, W4, b4, W5, b5, edge_index):
    raise NotImplementedError("write your pallas kernel here")


# speedup vs baseline: 5.9570x; 5.9570x over previous
import dataclasses
import functools

import jax
import jax.numpy as jnp
from jax import lax
from jax.experimental import pallas as pl
from jax.experimental.pallas import tpu as pltpu
from jax.experimental.pallas import tpu_sc as plsc

NNODES = 100000
NEDGES = 1600000

NC = 2
NS = 16
NW = NC * NS
LANES = 16

RSHIFT = 14
RS = 1 << RSHIFT
RMASK = RS - 1
NR = (NNODES + RS - 1) // RS
DUMP = RS
ACC_ROWS = RS + 16

BLK = 2048
NBLK = NEDGES // BLK
TAIL = NEDGES - NBLK * BLK
TAIL_TILE = NBLK % NW
KMAX = (NBLK + NW - 1) // NW
CAP = KMAX * BLK
FLUSH = 2048
STAGE_CAP = FLUSH + 128

WIN = 128
SUPW = 4
SW = SUPW * WIN


def _vector_mesh():
    return plsc.VectorSubcoreMesh(core_axis_name="c", subcore_axis_name="s")


def _sc_params():
    cp = pltpu.CompilerParams()
    fields = pltpu.CompilerParams.__dataclass_fields__
    if "needs_layout_passes" in fields:
        cp = dataclasses.replace(cp, needs_layout_passes=False)
    if "use_tc_tiling_on_sc" in fields:
        cp = dataclasses.replace(cp, use_tc_tiling_on_sc=False)
    return cp


def _bucket_edges(edge_flat):
    out_type = (
        jax.ShapeDtypeStruct((NW * NR * CAP,), jnp.int32),
        jax.ShapeDtypeStruct((NW * NR * CAP,), jnp.int32),
        jax.ShapeDtypeStruct((NW * 16,), jnp.int32),
    )

    @functools.partial(
        pl.kernel,
        out_type=out_type,
        mesh=_vector_mesh(),
        compiler_params=_sc_params(),
        scratch_types=[
            pltpu.VMEM((BLK,), jnp.int32),
            pltpu.VMEM((BLK,), jnp.int32),
            pltpu.VMEM((NR * STAGE_CAP,), jnp.int32),
            pltpu.VMEM((NR * STAGE_CAP,), jnp.int32),
            pltpu.VMEM((16,), jnp.int32),
            pltpu.SMEM((8,), jnp.int32),
            pltpu.SMEM((8,), jnp.int32),
        ],
    )
    def k(ei, bsrc, bdst, counts, eb_s, eb_d, st_s, st_d, cv, cnt_sm, fb_sm):
        cid = lax.axis_index("c")
        sid = lax.axis_index("s")
        tid = sid * NC + cid
        for r in range(8):
            cnt_sm[r] = 0
            fb_sm[r] = 0

        def do_block(i, size):
            off = pl.multiple_of(i * BLK, BLK)
            pltpu.sync_copy(ei.at[pl.ds(off, size)], eb_s.at[pl.ds(0, size)])
            pltpu.sync_copy(ei.at[pl.ds(pl.multiple_of(NEDGES + i * BLK, 512), size)],
                            eb_d.at[pl.ds(0, size)])

            @pl.loop(0, size, step=LANES)
            def _(j):
                sv = eb_s[pl.ds(j, LANES)]
                dv = eb_d[pl.ds(j, LANES)]
                bv = lax.shift_right_logical(dv, RSHIFT)
                locv = lax.bitwise_and(dv, RMASK)
                for r in range(NR):
                    m = bv == r
                    mi = m.astype(jnp.int32)
                    pref = plsc.cumsum(mi)
                    cnt = cnt_sm[r]
                    pos = (r * STAGE_CAP) + cnt + pref - 1
                    plsc.store_scatter(st_s, [pos], sv, mask=m)
                    plsc.store_scatter(st_d, [pos], locv, mask=m)
                    ncnt = cnt + jnp.sum(mi)
                    cnt_sm[r] = ncnt

                    @pl.when(ncnt >= FLUSH)
                    def _flush():
                        fb = fb_sm[r]
                        dst_off = pl.multiple_of((tid * NR + r) * CAP + fb, FLUSH)
                        pltpu.sync_copy(st_s.at[pl.ds(r * STAGE_CAP, FLUSH)],
                                        bsrc.at[pl.ds(dst_off, FLUSH)])
                        pltpu.sync_copy(st_d.at[pl.ds(r * STAGE_CAP, FLUSH)],
                                        bdst.at[pl.ds(dst_off, FLUSH)])
                        st_s[pl.ds(r * STAGE_CAP, LANES)] = (
                            st_s[pl.ds(r * STAGE_CAP + FLUSH, LANES)])
                        st_d[pl.ds(r * STAGE_CAP, LANES)] = (
                            st_d[pl.ds(r * STAGE_CAP + FLUSH, LANES)])
                        cnt_sm[r] = ncnt - FLUSH
                        fb_sm[r] = fb + FLUSH

        @pl.loop(0, KMAX)
        def _(kk):
            i = kk * NW + tid

            @pl.when(i < NBLK)
            def _():
                do_block(i, BLK)

        @pl.when(tid == TAIL_TILE)
        def _():
            do_block(NBLK, TAIL)

        tot = jnp.zeros((LANES,), jnp.int32)
        iot = lax.iota(jnp.int32, LANES)
        for r in range(NR):
            cnt = cnt_sm[r]
            fb = fb_sm[r]

            @pl.when(cnt > 0)
            def _final(r=r, fb=fb, cnt=cnt):
                dst_off = pl.multiple_of((tid * NR + r) * CAP + fb, FLUSH)
                pltpu.sync_copy(st_s.at[pl.ds(r * STAGE_CAP, FLUSH)],
                                bsrc.at[pl.ds(dst_off, FLUSH)])
                pltpu.sync_copy(st_d.at[pl.ds(r * STAGE_CAP, FLUSH)],
                                bdst.at[pl.ds(dst_off, FLUSH)])

            tot = jnp.where(iot == r, fb + cnt, tot)
        cv[...] = tot
        pltpu.sync_copy(cv, counts.at[pl.ds(pl.multiple_of(tid * 16, 16), 16)])

    return k(edge_flat)


def _sc_aggregate(sup, bsrc, bdst, counts):
    f = sup.shape[1]
    out_type = jax.ShapeDtypeStruct((2, NNODES, f), jnp.float32)

    @functools.partial(
        pl.kernel,
        out_type=out_type,
        mesh=_vector_mesh(),
        compiler_params=_sc_params(),
        scratch_types=[
            pltpu.VMEM_SHARED((ACC_ROWS, f), jnp.float32),
            pltpu.VMEM((128, f), jnp.float32),
            pltpu.VMEM((SW,), jnp.int32),
            pltpu.VMEM((SW,), jnp.int32),
            pltpu.VMEM((SUPW, WIN), jnp.int32),
            pltpu.VMEM((SUPW, WIN), jnp.int32),
            pltpu.VMEM((SUPW, WIN, f), jnp.float32),
            pltpu.VMEM((16,), jnp.int32),
            pltpu.SemaphoreType.DMA,
        ],
    )
    def k(sup_hbm, bs_hbm, bd_hbm, cnts_hbm, o_hbm,
          acc, zbuf, sv, dv, gi, si, rows, cv, gsem):
        cid = lax.axis_index("c")
        sid = lax.axis_index("s")
        t = 2 * sid + cid
        iot = lax.iota(jnp.int32, LANES)

        @pl.loop(0, 128)
        def _(i):
            @pl.loop(0, f, step=LANES)
            def _(j):
                zbuf[i, pl.ds(j, LANES)] = jnp.zeros((LANES,), jnp.float32)

        pltpu.sync_copy(cnts_hbm.at[pl.ds(pl.multiple_of(t * 16, 16), 16)], cv)
        cvv = cv[...]

        for r in range(NR):
            @pl.loop(0, RS // 128)
            def _(b):
                @pl.when(lax.rem(b, NS) == sid)
                def _():
                    pltpu.sync_copy(zbuf, acc.at[pl.ds(b * 128, 128)])

            plsc.subcore_barrier()

            cnt = jnp.sum(jnp.where(iot == r, cvv, 0))
            nsw = (cnt + (SW - 1)) // SW

            def swbody(w, carry, r=r):
                base = w * SW
                lbase = pl.multiple_of((t * NR + r) * CAP + base, SW)
                pltpu.sync_copy(bs_hbm.at[pl.ds(lbase, SW)], sv)
                pltpu.sync_copy(bd_hbm.at[pl.ds(lbase, SW)], dv)
                for u in range(SUPW):
                    @pl.loop(0, WIN, step=LANES)
                    def _(j, u=u):
                        pos = base + u * WIN + j + iot
                        valid = pos < cnt
                        gi[u, pl.ds(j, LANES)] = jnp.where(
                            valid, sv[pl.ds(u * WIN + j, LANES)], 0)
                        si[u, pl.ds(j, LANES)] = jnp.where(
                            valid, dv[pl.ds(u * WIN + j, LANES)], DUMP)
                cps = [pltpu.async_copy(sup_hbm.at[gi.at[u]], rows.at[u], gsem)
                       for u in range(SUPW)]
                for cp in cps:
                    cp.wait()
                for u in range(SUPW):
                    pltpu.sync_copy(rows.at[u], acc.at[si.at[u]], add=True)
                return carry

            lax.fori_loop(0, nsw, swbody, 0)
            plsc.subcore_barrier()

            rows_r = min(RS, NNODES - r * RS)
            nfull = rows_r // 128
            rem = rows_r - nfull * 128

            @pl.loop(0, nfull)
            def _(b, r=r):
                @pl.when(lax.rem(b, NS) == sid)
                def _():
                    pltpu.sync_copy(
                        acc.at[pl.ds(b * 128, 128)],
                        o_hbm.at[cid, pl.ds(r * RS + b * 128, 128)])

            if rem:
                @pl.when(sid == NS - 1)
                def _(r=r, nfull=nfull, rem=rem):
                    pltpu.sync_copy(
                        acc.at[pl.ds(nfull * 128, rem)],
                        o_hbm.at[cid, pl.ds(r * RS + nfull * 128, rem)])

            plsc.subcore_barrier()

    return k(sup, bsrc, bdst, counts)


def _leaky(x):
    return jnp.where(x > 0, x, 0.1 * x)


def _tc_stage(aggpair, pre, mats, out_dim, slice_out=False):
    fin = aggpair.shape[2]
    br = 1000
    grid = (NNODES // br,)

    in_specs = [pl.BlockSpec((2, br, fin), lambda i: (0, i, 0))]
    args = [aggpair]
    if pre is not None:
        b0, _ = pre
        in_specs.append(pl.BlockSpec((1, fin), lambda i: (0, 0)))
        args.append(b0.reshape(1, -1))
    for (w, bw, _aw) in mats:
        in_specs.append(pl.BlockSpec(w.shape, lambda i: (0, 0)))
        args.append(w)
        if bw is not None:
            in_specs.append(pl.BlockSpec((1, w.shape[1]), lambda i: (0, 0)))
            args.append(bw.reshape(1, -1))

    def body(*refs):
        oref = refs[-1]
        it = iter(refs[:-1])
        aref = next(it)
        t = aref[0] + aref[1]
        if pre is not None:
            t = t + next(it)[...]
            if pre[1]:
                t = _leaky(t)
        for (_w, bw, aw) in mats:
            t = jnp.dot(t, next(it)[...], preferred_element_type=jnp.float32)
            if bw is not None:
                t = t + next(it)[...]
            if aw:
                t = _leaky(t)
        if slice_out:
            t = t[:, :out_dim]
        oref[...] = t

    return pl.pallas_call(
        body,
        grid=grid,
        in_specs=in_specs,
        out_specs=pl.BlockSpec((br, out_dim), lambda i: (i, 0)),
        out_shape=jax.ShapeDtypeStruct((NNODES, out_dim), jnp.float32),
    )(*args)


def kernel(x, W1, b1, W2, b2, W3, b3, W4, b4, W5, b5, edge_index):
    bsrc, bdst, counts = _bucket_edges(edge_index.reshape(-1))

    a0 = _sc_aggregate(x, bsrc, bdst, counts)
    h1 = _tc_stage(a0, None, [(W1, b1, True)], 64)
    a1 = _sc_aggregate(h1, bsrc, bdst, counts)
    s3 = _tc_stage(a1, None, [(W2, b2, True), (W3, None, False)], 64)
    a3 = _sc_aggregate(s3, bsrc, bdst, counts)
    s4 = _tc_stage(a3, (b3, True), [(W4, None, False)], 32)
    a4 = _sc_aggregate(s4, bsrc, bdst, counts)
    W5p = jnp.pad(W5, ((0, 0), (0, 14)))
    s5 = _tc_stage(a4, (b4, True), [(W5p, None, False)], 16)
    a5 = _sc_aggregate(s5, bsrc, bdst, counts)
    out = _tc_stage(a5, (jnp.pad(b5, (0, 14)), False), [], 2, slice_out=True)
    return out


# speedup vs baseline: 10.4251x; 1.7500x over previous
import dataclasses
import functools

import jax
import jax.numpy as jnp
from jax import lax
from jax.experimental import pallas as pl
from jax.experimental.pallas import tpu as pltpu
from jax.experimental.pallas import tpu_sc as plsc

NNODES = 100000
NEDGES = 1600000

NC = 2
NS = 16
NW = NC * NS
LANES = 16

RSHIFT = 14
RS = 1 << RSHIFT
RMASK = RS - 1
NR = (NNODES + RS - 1) // RS
DUMP = RS
ACC_ROWS = RS + 16

BLK = 2048
NBLK = NEDGES // BLK
TAIL = NEDGES - NBLK * BLK
TAIL_TILE = NBLK % NW
KMAX = (NBLK + NW - 1) // NW
FLUSH = 2048
CAP = KMAX * BLK + FLUSH
STAGE_CAP = FLUSH + 128

WIN = 128
SUPW = 6
SW = SUPW * WIN


def _vector_mesh():
    return plsc.VectorSubcoreMesh(core_axis_name="c", subcore_axis_name="s")


def _sc_params():
    cp = pltpu.CompilerParams()
    fields = pltpu.CompilerParams.__dataclass_fields__
    if "needs_layout_passes" in fields:
        cp = dataclasses.replace(cp, needs_layout_passes=False)
    if "use_tc_tiling_on_sc" in fields:
        cp = dataclasses.replace(cp, use_tc_tiling_on_sc=False)
    return cp


def _bucket_edges(edge_flat):
    out_type = (
        jax.ShapeDtypeStruct((NW * NR * CAP,), jnp.int32),
        jax.ShapeDtypeStruct((NW * NR * CAP,), jnp.int32),
        jax.ShapeDtypeStruct((NW * 16,), jnp.int32),
    )

    @functools.partial(
        pl.kernel,
        out_type=out_type,
        mesh=_vector_mesh(),
        compiler_params=_sc_params(),
        scratch_types=[
            pltpu.VMEM((BLK,), jnp.int32),
            pltpu.VMEM((BLK,), jnp.int32),
            pltpu.VMEM((NR * STAGE_CAP,), jnp.int32),
            pltpu.VMEM((NR * STAGE_CAP,), jnp.int32),
            pltpu.VMEM((16,), jnp.int32),
            pltpu.SMEM((8,), jnp.int32),
            pltpu.SMEM((8,), jnp.int32),
        ],
    )
    def k(ei, bsrc, bdst, counts, eb_s, eb_d, st_s, st_d, cv, cnt_sm, fb_sm):
        cid = lax.axis_index("c")
        sid = lax.axis_index("s")
        tid = sid * NC + cid
        for r in range(8):
            cnt_sm[r] = 0
            fb_sm[r] = 0

        def do_block(i, size):
            off = pl.multiple_of(i * BLK, BLK)
            pltpu.sync_copy(ei.at[pl.ds(off, size)], eb_s.at[pl.ds(0, size)])
            pltpu.sync_copy(ei.at[pl.ds(pl.multiple_of(NEDGES + i * BLK, 512), size)],
                            eb_d.at[pl.ds(0, size)])

            @pl.loop(0, size, step=LANES)
            def _(j):
                sv = eb_s[pl.ds(j, LANES)]
                dv = eb_d[pl.ds(j, LANES)]
                bv = lax.shift_right_logical(dv, RSHIFT)
                locv = lax.bitwise_and(dv, RMASK)
                for r in range(NR):
                    m = bv == r
                    mi = m.astype(jnp.int32)
                    pref = plsc.cumsum(mi)
                    cnt = cnt_sm[r]
                    pos = (r * STAGE_CAP) + cnt + pref - 1
                    plsc.store_scatter(st_s, [pos], sv, mask=m)
                    plsc.store_scatter(st_d, [pos], locv, mask=m)
                    ncnt = cnt + jnp.sum(mi)
                    cnt_sm[r] = ncnt

                    @pl.when(ncnt >= FLUSH)
                    def _flush():
                        fb = fb_sm[r]
                        dst_off = pl.multiple_of((tid * NR + r) * CAP + fb, FLUSH)
                        pltpu.sync_copy(st_s.at[pl.ds(r * STAGE_CAP, FLUSH)],
                                        bsrc.at[pl.ds(dst_off, FLUSH)])
                        pltpu.sync_copy(st_d.at[pl.ds(r * STAGE_CAP, FLUSH)],
                                        bdst.at[pl.ds(dst_off, FLUSH)])
                        st_s[pl.ds(r * STAGE_CAP, LANES)] = (
                            st_s[pl.ds(r * STAGE_CAP + FLUSH, LANES)])
                        st_d[pl.ds(r * STAGE_CAP, LANES)] = (
                            st_d[pl.ds(r * STAGE_CAP + FLUSH, LANES)])
                        cnt_sm[r] = ncnt - FLUSH
                        fb_sm[r] = fb + FLUSH

        @pl.loop(0, KMAX)
        def _(kk):
            i = kk * NW + tid

            @pl.when(i < NBLK)
            def _():
                do_block(i, BLK)

        @pl.when(tid == TAIL_TILE)
        def _():
            do_block(NBLK, TAIL)

        tot = jnp.zeros((LANES,), jnp.int32)
        iot = lax.iota(jnp.int32, LANES)
        for r in range(NR):
            cnt = cnt_sm[r]
            fb = fb_sm[r]

            @pl.when(cnt > 0)
            def _final(r=r, fb=fb, cnt=cnt):
                dst_off = pl.multiple_of((tid * NR + r) * CAP + fb, FLUSH)
                pltpu.sync_copy(st_s.at[pl.ds(r * STAGE_CAP, FLUSH)],
                                bsrc.at[pl.ds(dst_off, FLUSH)])
                pltpu.sync_copy(st_d.at[pl.ds(r * STAGE_CAP, FLUSH)],
                                bdst.at[pl.ds(dst_off, FLUSH)])

            tot = jnp.where(iot == r, fb + cnt, tot)
        cv[...] = tot
        pltpu.sync_copy(cv, counts.at[pl.ds(pl.multiple_of(tid * 16, 16), 16)])

    return k(edge_flat)


def _sc_aggregate(sup, bsrc, bdst, counts):
    f = sup.shape[1]
    out_type = jax.ShapeDtypeStruct((2, NNODES, f), jnp.float32)

    @functools.partial(
        pl.kernel,
        out_type=out_type,
        mesh=_vector_mesh(),
        compiler_params=_sc_params(),
        scratch_types=[
            pltpu.VMEM_SHARED((ACC_ROWS, f), jnp.float32),
            pltpu.VMEM((2, SW), jnp.int32),
            pltpu.VMEM((2, SW), jnp.int32),
            pltpu.VMEM((SUPW, WIN), jnp.int32),
            pltpu.VMEM((SUPW, WIN), jnp.int32),
            pltpu.VMEM((SUPW, WIN, f), jnp.float32),
            pltpu.VMEM((16,), jnp.int32),
            pltpu.SemaphoreType.DMA,
            pltpu.SemaphoreType.DMA,
        ],
    )
    def k(sup_hbm, bs_hbm, bd_hbm, cnts_hbm, o_hbm,
          acc, sv, dv, gi, si, rows, cv, gsem, isem):
        cid = lax.axis_index("c")
        sid = lax.axis_index("s")
        t = 2 * sid + cid
        iot = lax.iota(jnp.int32, LANES)

        pltpu.sync_copy(cnts_hbm.at[pl.ds(pl.multiple_of(t * 16, 16), 16)], cv)
        cvv = cv[...]

        for r in range(NR):
            @pl.loop(0, WIN)
            def _(i):
                @pl.loop(0, f, step=LANES)
                def _(j):
                    rows[0, i, pl.ds(j, LANES)] = jnp.zeros((LANES,),
                                                            jnp.float32)

            @pl.loop(0, RS // WIN)
            def _(b):
                @pl.when(lax.rem(b, NS) == sid)
                def _():
                    pltpu.sync_copy(rows.at[0], acc.at[pl.ds(b * WIN, WIN)])

            plsc.subcore_barrier()

            cnt = jnp.sum(jnp.where(iot == r, cvv, 0))
            nbulk = cnt // SW
            remn = cnt - nbulk * SW
            listbase = (t * NR + r) * CAP

            def _idx_refs(sw, buf, r=r):
                lb = pl.multiple_of(listbase + sw * SW, 256)
                return ((bs_hbm.at[pl.ds(lb, SW)], sv.at[buf]),
                        (bd_hbm.at[pl.ds(lb, SW)], dv.at[buf]))

            def issue_idx(sw, buf):
                for s_ref, d_ref in _idx_refs(sw, buf):
                    pltpu.async_copy(s_ref, d_ref, isem)

            def wait_idx(sw, buf):
                for s_ref, d_ref in _idx_refs(sw, buf):
                    pltpu.make_async_copy(s_ref, d_ref, isem).wait()

            @pl.when(nbulk > 0)
            def _():
                issue_idx(0, 0)

            def swbody(w2, carry, r=r):
                for half in (0, 1):
                    sw = 2 * w2 + half

                    @pl.when(sw < nbulk)
                    def _(sw=sw, half=half):
                        wait_idx(sw, half)
                        for u in range(SUPW):
                            @pl.loop(0, WIN, step=LANES)
                            def _(j, u=u, half=half):
                                si[u, pl.ds(j, LANES)] = (
                                    dv[half, pl.ds(u * WIN + j, LANES)])
                        cps = [pltpu.async_copy(
                                   sup_hbm.at[sv.at[half, pl.ds(u * WIN, WIN)]],
                                   rows.at[u], gsem)
                               for u in range(SUPW)]

                        @pl.when(sw + 1 < nbulk)
                        def _():
                            issue_idx(sw + 1, 1 - half)

                        for u in range(SUPW):
                            cps[u].wait()
                            pltpu.sync_copy(rows.at[u], acc.at[si.at[u]],
                                            add=True)
                return carry

            lax.fori_loop(0, (nbulk + 1) // 2, swbody, 0)

            @pl.when(remn > 0)
            def _(r=r):
                base = nbulk * SW
                lb = pl.multiple_of(listbase + base, 256)
                pltpu.sync_copy(bs_hbm.at[pl.ds(lb, SW)], sv.at[0])
                pltpu.sync_copy(bd_hbm.at[pl.ds(lb, SW)], dv.at[0])
                for u in range(SUPW):
                    @pl.when(u * WIN < remn)
                    def _(u=u):
                        @pl.loop(0, WIN, step=LANES)
                        def _(j, u=u):
                            pos = u * WIN + j + iot
                            valid = pos < remn
                            gi[u, pl.ds(j, LANES)] = jnp.where(
                                valid, sv[0, pl.ds(u * WIN + j, LANES)], 0)
                            si[u, pl.ds(j, LANES)] = jnp.where(
                                valid, dv[0, pl.ds(u * WIN + j, LANES)], DUMP)
                        pltpu.async_copy(sup_hbm.at[gi.at[u]], rows.at[u], gsem)
                for u in range(SUPW):
                    @pl.when(u * WIN < remn)
                    def _(u=u):
                        pltpu.make_async_copy(sup_hbm.at[gi.at[u]],
                                              rows.at[u], gsem).wait()
                        pltpu.sync_copy(rows.at[u], acc.at[si.at[u]], add=True)

            plsc.subcore_barrier()

            rows_r = min(RS, NNODES - r * RS)
            nfull = rows_r // 128
            rem = rows_r - nfull * 128

            @pl.loop(0, nfull)
            def _(b, r=r):
                @pl.when(lax.rem(b, NS) == sid)
                def _():
                    pltpu.sync_copy(
                        acc.at[pl.ds(b * 128, 128)],
                        o_hbm.at[cid, pl.ds(r * RS + b * 128, 128)])

            if rem:
                @pl.when(sid == NS - 1)
                def _(r=r, nfull=nfull, rem=rem):
                    pltpu.sync_copy(
                        acc.at[pl.ds(nfull * 128, rem)],
                        o_hbm.at[cid, pl.ds(r * RS + nfull * 128, rem)])

            plsc.subcore_barrier()

    return k(sup, bsrc, bdst, counts)


def _leaky(x):
    return jnp.where(x > 0, x, 0.1 * x)


def _tc_stage(aggpair, pre, mats, out_dim, slice_out=False):
    fin = aggpair.shape[2]
    br = 1000
    grid = (NNODES // br,)

    in_specs = [pl.BlockSpec((2, br, fin), lambda i: (0, i, 0))]
    args = [aggpair]
    if pre is not None:
        b0, _ = pre
        in_specs.append(pl.BlockSpec((1, fin), lambda i: (0, 0)))
        args.append(b0.reshape(1, -1))
    for (w, bw, _aw) in mats:
        in_specs.append(pl.BlockSpec(w.shape, lambda i: (0, 0)))
        args.append(w)
        if bw is not None:
            in_specs.append(pl.BlockSpec((1, w.shape[1]), lambda i: (0, 0)))
            args.append(bw.reshape(1, -1))

    def body(*refs):
        oref = refs[-1]
        it = iter(refs[:-1])
        aref = next(it)
        t = aref[0] + aref[1]
        if pre is not None:
            t = t + next(it)[...]
            if pre[1]:
                t = _leaky(t)
        for (_w, bw, aw) in mats:
            t = jnp.dot(t, next(it)[...], preferred_element_type=jnp.float32)
            if bw is not None:
                t = t + next(it)[...]
            if aw:
                t = _leaky(t)
        if slice_out:
            t = t[:, :out_dim]
        oref[...] = t

    return pl.pallas_call(
        body,
        grid=grid,
        in_specs=in_specs,
        out_specs=pl.BlockSpec((br, out_dim), lambda i: (i, 0)),
        out_shape=jax.ShapeDtypeStruct((NNODES, out_dim), jnp.float32),
    )(*args)


def kernel(x, W1, b1, W2, b2, W3, b3, W4, b4, W5, b5, edge_index):
    bsrc, bdst, counts = _bucket_edges(edge_index.reshape(-1))

    a0 = _sc_aggregate(x, bsrc, bdst, counts)
    h1 = _tc_stage(a0, None, [(W1, b1, True)], 64)
    a1 = _sc_aggregate(h1, bsrc, bdst, counts)
    s3 = _tc_stage(a1, None, [(W2, b2, True), (W3, None, False)], 64)
    a3 = _sc_aggregate(s3, bsrc, bdst, counts)
    s4 = _tc_stage(a3, (b3, True), [(W4, None, False)], 32)
    a4 = _sc_aggregate(s4, bsrc, bdst, counts)
    W5p = jnp.pad(W5, ((0, 0), (0, 14)))
    s5 = _tc_stage(a4, (b4, True), [(W5p, None, False)], 16)
    a5 = _sc_aggregate(s5, bsrc, bdst, counts)
    out = _tc_stage(a5, (jnp.pad(b5, (0, 14)), False), [], 2, slice_out=True)
    return out
